# baseline (speedup 1.0000x reference)
import functools

import jax
import jax.numpy as jnp
from jax import lax
from jax.experimental import pallas as pl
from jax.experimental.pallas import tpu as pltpu
from jax.experimental.pallas import tpu_sc as plsc

CUSTOMER_VOCAB = 1000001
AGE_VOCAB = 101
EMBED_DIM = 32
BATCH = 16384
OUT_W = 2 * EMBED_DIM

_INFO = plsc.get_sparse_core_info()
_NC = _INFO.num_cores
_NS = _INFO.num_subcores
_NW = _NC * _NS
_BPW = BATCH // _NW

_CHUNK_COLS = 768
_NFULL = CUSTOMER_VOCAB // _CHUNK_COLS
_TAILW = CUSTOMER_VOCAB - _NFULL * _CHUNK_COLS
_NCHUNKS = _NFULL + 1
_CPW = (_NCHUNKS + _NW - 1) // _NW
_POS_BITS = 14
_DUMP = BATCH * OUT_W

_mesh = plsc.VectorSubcoreMesh(core_axis_name="c", subcore_axis_name="s")


@functools.partial(
    pl.kernel,
    mesh=_mesh,
    out_type=jax.ShapeDtypeStruct((BATCH * OUT_W + _NW * OUT_W,), jnp.float32),
    scratch_types=[
        pltpu.VMEM((BATCH,), jnp.int32),
        pltpu.VMEM((_BPW,), jnp.int32),
        pltpu.VMEM((BATCH + 16,), jnp.int32),
        pltpu.VMEM((EMBED_DIM, _CHUNK_COLS), jnp.float32),
        pltpu.VMEM((EMBED_DIM, 1024), jnp.float32),
        pltpu.VMEM((80,), jnp.int32),
        pltpu.VMEM((1, 16 * EMBED_DIM), jnp.float32),
        pltpu.VMEM((1, 16 * EMBED_DIM), jnp.int32),
        pltpu.SemaphoreType.DMA,
    ],
    compiler_params=pltpu.CompilerParams(needs_layout_passes=False),
)
def _embed_concat(cust_hbm, age_hbm, tabT_hbm, ageT_hbm, tailT_hbm, out_hbm,
                  ids_v, age_v, my_pk, buf, resbuf, hits, vals, idxs, sem):
    wid = lax.axis_index("s") * _NC + lax.axis_index("c")
    base = wid * _BPW
    lane = lax.iota(jnp.int32, 16)

    pltpu.sync_copy(cust_hbm, ids_v)
    pltpu.sync_copy(age_hbm.at[pl.ds(base, _BPW)], age_v)
    pltpu.sync_copy(ageT_hbm, resbuf.at[:, pl.ds(0, 128)])
    pltpu.sync_copy(tailT_hbm, resbuf.at[:, pl.ds(128, 128)])

    def age_group(g, carry):
        avec = age_v[pl.ds(g * 16, 16)]
        posv = (base + g * 16 + lane) * OUT_W
        for d in range(EMBED_DIM):
            v = plsc.load_gather(resbuf,
                                 [jnp.full((16,), d, jnp.int32), avec])
            slot = lane * EMBED_DIM + d
            plsc.store_scatter(vals.at[0], [slot], v)
            plsc.store_scatter(idxs.at[0], [slot], posv + (EMBED_DIM + d))
        pltpu.async_copy(vals.at[0], out_hbm.at[idxs.at[0]], sem).wait()
        return carry

    lax.fori_loop(0, _BPW // 16, age_group, jnp.int32(0))

    lo = wid * _CPW
    col0 = lo * _CHUNK_COLS

    def scan_body(g, cnt):
        idv = ids_v[pl.ds(g * 16, 16)]
        ch = lax.div(idv, jnp.int32(_CHUNK_COLS))
        mask = (ch >= lo) & (ch < lo + _CPW)
        n = plsc.all_reduce_population_count(mask)
        packed = ((idv - col0) << _POS_BITS) | (g * 16 + lane)
        plsc.store_compressed(my_pk.at[pl.ds(cnt, 16)], packed, mask=mask)
        return cnt + n[0]

    cnt = lax.fori_loop(0, BATCH // 16, scan_body, jnp.int32(0))

    def serve(src, hs, k, src_col0, m):
        h = hits[pl.ds(hs, 16)]
        hpos = h & ((1 << _POS_BITS) - 1)
        local = (h >> _POS_BITS) - k * _CHUNK_COLS + src_col0
        local = jnp.where(m, local, 0)
        dump = _DUMP + wid * OUT_W
        for d in range(EMBED_DIM):
            v = plsc.load_gather(src, [jnp.full((16,), d, jnp.int32), local],
                                 mask=m)
            slot = lane * EMBED_DIM + d
            plsc.store_scatter(vals.at[0], [slot], v)
            tgt = jnp.where(m, hpos * OUT_W + d, dump + d)
            plsc.store_scatter(idxs.at[0], [slot], tgt)
        pltpu.async_copy(vals.at[0], out_hbm.at[idxs.at[0]], sem).wait()

    full16 = jnp.full((16,), True)

    def process_chunk(k, src, src_col0):
        nsteps = lax.div(cnt + 15, jnp.int32(16))

        def step(j, hc):
            lm = j * 16 + lane < cnt
            h = my_pk[pl.ds(j * 16, 16)]
            rel = h >> _POS_BITS
            inm = lm & (lax.div(rel, jnp.int32(_CHUNK_COLS)) == k)
            n = plsc.all_reduce_population_count(inm)
            plsc.store_compressed(hits.at[pl.ds(hc, 16)], h, mask=inm)
            hc = hc + n[0]

            @pl.when(hc >= 16)
            def _():
                serve(src, hc - 16, k, src_col0, full16)

            return jnp.where(hc >= 16, hc - 16, hc)

        hc = lax.fori_loop(0, nsteps, step, jnp.int32(0))

        @pl.when(hc >= 16)
        def _():
            serve(src, hc - 16, k, src_col0, full16)

        hc = jnp.where(hc >= 16, hc - 16, hc)

        @pl.when(hc > 0)
        def _():
            serve(src, jnp.int32(0), k, src_col0, lane < hc)

    def chunk_body(k, carry):
        chunk = wid * _CPW + k

        @pl.when(chunk < _NFULL)
        def _():
            start = pl.multiple_of(chunk * _CHUNK_COLS, 128)
            pltpu.sync_copy(tabT_hbm.at[:, pl.ds(start, _CHUNK_COLS)], buf)
            process_chunk(k, buf, jnp.int32(0))

        return carry

    lax.fori_loop(0, _CPW, chunk_body, jnp.int32(0))

    @pl.when(wid == _NW - 1)
    def _():
        k = jnp.int32(_NFULL - lo)
        process_chunk(k, resbuf, jnp.int32(128))


def kernel(customer_id, age, customer_table, age_table):
    tail = jnp.pad(customer_table[_NFULL * _CHUNK_COLS:].T,
                   ((0, 0), (0, 128 - _TAILW)))
    agep = jnp.pad(age_table.T, ((0, 0), (0, 128 - AGE_VOCAB)))
    flat = _embed_concat(customer_id, age, customer_table.T, agep, tail)
    return flat[:BATCH * OUT_W].reshape(BATCH, OUT_W)
